# baseline (speedup 1.0000x reference)
import functools

import jax
import jax.numpy as jnp
from jax.experimental import pallas as pl
from jax.experimental.pallas import tpu as pltpu

N_TOKENS = 8192
D_MODEL = 2048
NUM_EXPERTS = 64
TOP_K = 8
BLOCK_R = 2048
BLOCK_K = 512


def _router_body(x_ref, w_ref, scores_ref, wts_ref, idx_ref, cnt_ref,
                 acc_ref):
    k = pl.program_id(1)
    nk = pl.num_programs(1)
    part = jnp.dot(x_ref[...], w_ref[...], preferred_element_type=jnp.float32)

    @pl.when(k == 0)
    def _first():
        acc_ref[...] = part

    @pl.when(k > 0)
    def _accum():
        acc_ref[...] += part

    @pl.when((pl.program_id(0) == 0) & (k == 0))
    def _init():
        cnt_ref[...] = jnp.zeros_like(cnt_ref)

    @pl.when(k == nk - 1)
    def _epilogue():
        logits = acc_ref[...]
        m = jnp.max(logits, axis=-1, keepdims=True)
        e = jnp.exp(logits - m)
        scores_ref[...] = e / jnp.sum(e, axis=-1, keepdims=True)

        lt = jnp.transpose(logits)
        mt = jnp.max(lt, axis=0, keepdims=True)
        st = jnp.sum(jnp.exp(lt - mt), axis=0, keepdims=True)

        row = jax.lax.broadcasted_iota(jnp.int32, (NUM_EXPERTS, BLOCK_R), 0)
        work = lt
        wrows, irows = [], []
        for _ in range(TOP_K):
            mk = jnp.max(work, axis=0, keepdims=True)
            sel = jnp.min(jnp.where(work == mk, row, NUM_EXPERTS), axis=0,
                          keepdims=True)
            work = jnp.where(row == sel, -jnp.inf, work)
            wrows.append(jnp.exp(mk - mt) / st)
            irows.append(sel)
        wts_ref[...] = jnp.transpose(jnp.concatenate(wrows, axis=0))
        idx_ref[...] = jnp.transpose(jnp.concatenate(irows, axis=0))

        cnt_ref[...] += jnp.sum((work == -jnp.inf).astype(jnp.float32),
                                axis=1, keepdims=True)


@functools.partial(jax.jit, static_argnames=("interpret",))
def _router(x, W, interpret=False):
    grid = (N_TOKENS // BLOCK_R, D_MODEL // BLOCK_K)
    scores, wts, idx, cnt = pl.pallas_call(
        _router_body,
        grid=grid,
        in_specs=[
            pl.BlockSpec((BLOCK_R, BLOCK_K), lambda i, k: (i, k)),
            pl.BlockSpec((BLOCK_K, NUM_EXPERTS), lambda i, k: (k, 0)),
        ],
        out_specs=[
            pl.BlockSpec((BLOCK_R, NUM_EXPERTS), lambda i, k: (i, 0)),
            pl.BlockSpec((BLOCK_R, TOP_K), lambda i, k: (i, 0)),
            pl.BlockSpec((BLOCK_R, TOP_K), lambda i, k: (i, 0)),
            pl.BlockSpec((NUM_EXPERTS, 1), lambda i, k: (0, 0)),
        ],
        out_shape=[
            jax.ShapeDtypeStruct((N_TOKENS, NUM_EXPERTS), jnp.float32),
            jax.ShapeDtypeStruct((N_TOKENS, TOP_K), jnp.float32),
            jax.ShapeDtypeStruct((N_TOKENS, TOP_K), jnp.int32),
            jax.ShapeDtypeStruct((NUM_EXPERTS, 1), jnp.float32),
        ],
        scratch_shapes=[pltpu.VMEM((BLOCK_R, NUM_EXPERTS), jnp.float32)],
        interpret=interpret,
    )(x, W)
    return scores, wts, idx, cnt.reshape(NUM_EXPERTS)


def kernel(x, W):
    return _router(x, W)


# speedup vs baseline: 1.2671x; 1.2671x over previous
import functools

import jax
import jax.numpy as jnp
from jax.experimental import pallas as pl

N_TOKENS = 8192
D_MODEL = 2048
NUM_EXPERTS = 64
TOP_K = 8
BLOCK_R = 2048


def _router_body(x_ref, w_ref, scores_ref, wts_ref, idx_ref, cnt_ref):
    logits = jnp.dot(x_ref[...], w_ref[...], preferred_element_type=jnp.float32)
    m = jnp.max(logits, axis=-1, keepdims=True)
    e = jnp.exp(logits - m)
    scores_ref[...] = e / jnp.sum(e, axis=-1, keepdims=True)

    lt = jnp.transpose(logits)
    mt = jnp.max(lt, axis=0, keepdims=True)
    st = jnp.sum(jnp.exp(lt - mt), axis=0, keepdims=True)

    row = jax.lax.broadcasted_iota(jnp.int32, (NUM_EXPERTS, BLOCK_R), 0)
    work = lt
    wrows, irows = [], []
    for _ in range(TOP_K):
        mk = jnp.max(work, axis=0, keepdims=True)
        sel = jnp.min(jnp.where(work == mk, row, NUM_EXPERTS), axis=0,
                      keepdims=True)
        work = jnp.where(row == sel, -jnp.inf, work)
        wrows.append(jnp.exp(mk - mt) / st)
        irows.append(sel)
    wts_t = jnp.concatenate(wrows, axis=0)
    idx_t = jnp.concatenate(irows, axis=0)
    wts_ref[...] = jnp.transpose(wts_t)
    idx_ref[...] = jnp.transpose(idx_t)

    @pl.when(pl.program_id(0) == 0)
    def _init():
        cnt_ref[...] = jnp.zeros_like(cnt_ref)

    cnt_ref[...] += jnp.sum((work == -jnp.inf).astype(jnp.float32), axis=1,
                            keepdims=True)


@functools.partial(jax.jit, static_argnames=("interpret",))
def _router(x, W, interpret=False):
    grid = N_TOKENS // BLOCK_R
    scores, wts, idx, cnt = pl.pallas_call(
        _router_body,
        grid=(grid,),
        in_specs=[
            pl.BlockSpec((BLOCK_R, D_MODEL), lambda i: (i, 0)),
            pl.BlockSpec((D_MODEL, NUM_EXPERTS), lambda i: (0, 0)),
        ],
        out_specs=[
            pl.BlockSpec((BLOCK_R, NUM_EXPERTS), lambda i: (i, 0)),
            pl.BlockSpec((BLOCK_R, TOP_K), lambda i: (i, 0)),
            pl.BlockSpec((BLOCK_R, TOP_K), lambda i: (i, 0)),
            pl.BlockSpec((NUM_EXPERTS, 1), lambda i: (0, 0)),
        ],
        out_shape=[
            jax.ShapeDtypeStruct((N_TOKENS, NUM_EXPERTS), jnp.float32),
            jax.ShapeDtypeStruct((N_TOKENS, TOP_K), jnp.float32),
            jax.ShapeDtypeStruct((N_TOKENS, TOP_K), jnp.int32),
            jax.ShapeDtypeStruct((NUM_EXPERTS, 1), jnp.float32),
        ],
        interpret=interpret,
    )(x, W)
    return scores, wts, idx, cnt.reshape(NUM_EXPERTS)


def kernel(x, W):
    return _router(x, W)
